# baseline (speedup 1.0000x reference)
import functools

import jax
import jax.numpy as jnp
from jax import lax
from jax.experimental import pallas as pl
from jax.experimental.pallas import tpu as pltpu
from jax.experimental.pallas import tpu_sc as plsc

DIM = 128
QUEUE_SIZE = 65536
BATCH_COLS = 4096

_ROWS_W = 8
_HCOLS = BATCH_COLS // 2


def _sc_body(lk_ref, out_ref, buf):
    cid = lax.axis_index("c")
    sid = lax.axis_index("s")
    wid = sid * 2 + cid
    r = wid // 2
    h = wid % 2
    row0 = r * _ROWS_W
    col0 = h * _HCOLS
    pltpu.sync_copy(lk_ref.at[pl.ds(row0, _ROWS_W), pl.ds(col0, _HCOLS)], buf)
    pltpu.sync_copy(buf, out_ref.at[pl.ds(row0, _ROWS_W), pl.ds(col0, _HCOLS)])


def _sc_head_full(last_k):
    mesh = plsc.VectorSubcoreMesh(core_axis_name="c", subcore_axis_name="s")
    run = functools.partial(
        pl.kernel,
        out_type=jax.ShapeDtypeStruct((DIM, QUEUE_SIZE), jnp.float32),
        mesh=mesh,
        scratch_types=[pltpu.VMEM((_ROWS_W, _HCOLS), jnp.float32)],
    )(_sc_body)
    return run(last_k)


_TW = 7680
_TCHUNK = (QUEUE_SIZE - BATCH_COLS) // _TW
_TNBUF = 4


def _tc_body(q_ref, tmp_ref, out_ref, buf, rsem, wsem):
    def rd(c):
        b = c % _TNBUF
        return pltpu.make_async_copy(
            q_ref.at[:, pl.ds(BATCH_COLS + c * _TW, _TW)], buf.at[b], rsem.at[b]
        )

    def wr(c):
        b = c % _TNBUF
        return pltpu.make_async_copy(
            buf.at[b], out_ref.at[:, pl.ds(BATCH_COLS + c * _TW, _TW)], wsem.at[b]
        )

    for c in range(_TNBUF):
        rd(c).start()
    for c in range(_TCHUNK):
        rd(c).wait()
        wr(c).start()
        if c + _TNBUF < _TCHUNK:
            wr(c).wait()
            rd(c + _TNBUF).start()
    for c in range(max(_TCHUNK - _TNBUF, 0), _TCHUNK):
        wr(c).wait()


def _tc_tail_inplace(moco_queue, tmp):
    return pl.pallas_call(
        _tc_body,
        in_specs=[
            pl.BlockSpec(memory_space=pl.ANY),
            pl.BlockSpec(memory_space=pl.ANY),
        ],
        out_specs=pl.BlockSpec(memory_space=pl.ANY),
        out_shape=jax.ShapeDtypeStruct((DIM, QUEUE_SIZE), jnp.float32),
        input_output_aliases={1: 0},
        scratch_shapes=[
            pltpu.VMEM((_TNBUF, DIM, _TW), jnp.float32),
            pltpu.SemaphoreType.DMA((_TNBUF,)),
            pltpu.SemaphoreType.DMA((_TNBUF,)),
        ],
    )(moco_queue, tmp)


def kernel(last_k, moco_queue):
    tmp = _sc_head_full(last_k)
    return _tc_tail_inplace(moco_queue, tmp)


# speedup vs baseline: 1.9649x; 1.9649x over previous
import jax
import jax.numpy as jnp
from jax.experimental import pallas as pl
from jax.experimental.pallas import tpu as pltpu

DIM = 128
QUEUE_SIZE = 65536
BATCH_COLS = 4096

_W = 8192
_NCHUNK = QUEUE_SIZE // _W
_NBUF = 6


def _ring_body(lk_ref, q_ref, out_ref, buf, rsem, wsem):
    def read_descs(c):
        b = c % _NBUF
        if c == 0:
            return [
                pltpu.make_async_copy(lk_ref, buf.at[b, :, pl.ds(0, BATCH_COLS)], rsem.at[b]),
                pltpu.make_async_copy(
                    q_ref.at[:, pl.ds(BATCH_COLS, _W - BATCH_COLS)],
                    buf.at[b, :, pl.ds(BATCH_COLS, _W - BATCH_COLS)],
                    rsem.at[b],
                ),
            ]
        return [
            pltpu.make_async_copy(
                q_ref.at[:, pl.ds(c * _W, _W)], buf.at[b], rsem.at[b]
            )
        ]

    def write_desc(c):
        b = c % _NBUF
        return pltpu.make_async_copy(
            buf.at[b], out_ref.at[:, pl.ds(c * _W, _W)], wsem.at[b]
        )

    for c in range(_NBUF):
        for d in read_descs(c):
            d.start()
    for c in range(_NCHUNK):
        for d in read_descs(c):
            d.wait()
        write_desc(c).start()
        if c + _NBUF < _NCHUNK:
            write_desc(c).wait()
            for d in read_descs(c + _NBUF):
                d.start()
    for c in range(max(_NCHUNK - _NBUF, 0), _NCHUNK):
        write_desc(c).wait()


def kernel(last_k, moco_queue):
    return pl.pallas_call(
        _ring_body,
        in_specs=[
            pl.BlockSpec(memory_space=pl.ANY),
            pl.BlockSpec(memory_space=pl.ANY),
        ],
        out_specs=pl.BlockSpec(memory_space=pl.ANY),
        out_shape=jax.ShapeDtypeStruct((DIM, QUEUE_SIZE), jnp.float32),
        scratch_shapes=[
            pltpu.VMEM((_NBUF, DIM, _W), jnp.float32),
            pltpu.SemaphoreType.DMA((_NBUF,)),
            pltpu.SemaphoreType.DMA((_NBUF,)),
        ],
    )(last_k, moco_queue)
